# baseline (speedup 1.0000x reference)
import functools

import jax
import jax.numpy as jnp
from jax import lax
from jax.experimental import pallas as pl
from jax.experimental.pallas import tpu as pltpu
from jax.experimental.pallas import tpu_sc as plsc

NUM_CORES = 2
NUM_SUBCORES = 16
NUM_WORKERS = NUM_CORES * NUM_SUBCORES

V = 1000000
D = 64
TILE_T = 128
NTB = (V + TILE_T - 1) // TILE_T
V_PAD = NTB * TILE_T
FULL_BLOCKS = V // TILE_T
TAIL0 = FULL_BLOCKS * TILE_T

A_PER_W = 246
A_START = 245

SEQ = 4096
TLEN = 200
B_TOTAL = SEQ * TLEN
SBLK = SEQ // TILE_T


def _idx_consts():
    i = lax.iota(jnp.int32, 16)
    return i >> 3, i & 7


def _splat(x):
    return jnp.full((16,), x, jnp.int32)


@functools.lru_cache(maxsize=None)
def _build_detile():
    mesh = plsc.VectorSubcoreMesh(core_axis_name="c", subcore_axis_name="s")

    @functools.partial(
        pl.kernel,
        out_type=jax.ShapeDtypeStruct((V_PAD * D,), jnp.float32),
        mesh=mesh,
        compiler_params=pltpu.CompilerParams(
            use_tc_tiling_on_sc=True, needs_layout_passes=False,
            disable_bounds_checks=True),
        scratch_types=[
            pltpu.VMEM((2, D, TILE_T), jnp.float32),
            pltpu.VMEM((TILE_T * D,), jnp.float32),
            pltpu.VMEM((TILE_T * D,), jnp.float32),
            pltpu.VMEM((D * D,), jnp.float32),
            pltpu.SemaphoreType.DMA,
            pltpu.SemaphoreType.DMA,
            pltpu.SemaphoreType.DMA,
            pltpu.SemaphoreType.DMA,
        ],
    )
    def detile(wt_hbm, tail_hbm, wpad_hbm, src_v, dst_v0, dst_v1, tail_v,
               gsem0, gsem1, osem0, osem1):
        gsems = (gsem0, gsem1)
        osems = (osem0, osem1)
        dsts = (dst_v0, dst_v1)
        wid = lax.axis_index("s") * NUM_CORES + lax.axis_index("c")
        start = wid * A_START
        C0, C1 = _idx_consts()

        def blk(i):
            return jnp.minimum(start + i, FULL_BLOCKS - 1)

        def fetch(i, b):
            pltpu.async_copy(
                wt_hbm.at[:, pl.ds(blk(i) * TILE_T, TILE_T)],
                src_v.at[b], gsems[b])

        def wait_fetch(b):
            pltpu.make_async_copy(
                wt_hbm.at[:, pl.ds(0, TILE_T)], src_v.at[b],
                gsems[b]).wait()

        lane = lax.iota(jnp.int32, 16)
        diag = [(lane + k) & 15 for k in range(16)]
        dstc = [d * D + lane for d in diag]

        def shuffle(b):
            @plsc.parallel_loop(0, TILE_T // 16)
            def _tblk(m):
                t0 = m * 16
                for f0 in range(0, D, 16):
                    ff = _splat(f0) + lane
                    for k in range(16):
                        v = plsc.load_gather(
                            src_v.at[b], [ff, _splat(t0) + diag[k]])
                        plsc.store_scatter(
                            dsts[b], [_splat(t0 * D + f0) + dstc[k]], v)

        def put(i, b):
            pltpu.async_copy(
                dsts[b],
                wpad_hbm.at[pl.ds(blk(i) * (TILE_T * D), TILE_T * D)],
                osems[b])

        def wait_put(b):
            pltpu.make_async_copy(
                dsts[b], wpad_hbm.at[pl.ds(0, TILE_T * D)],
                osems[b]).wait()

        fetch(0, 0)
        fetch(1, 1)
        wait_fetch(0)
        shuffle(0)
        put(0, 0)
        fetch(2, 0)
        wait_fetch(1)
        shuffle(1)
        put(1, 1)
        fetch(3, 1)

        @pl.loop(1, A_PER_W // 2 - 1)
        def _pair(k):
            i0 = 2 * k
            wait_put(0)
            wait_fetch(0)
            shuffle(0)
            put(i0, 0)
            fetch(i0 + 2, 0)
            wait_put(1)
            wait_fetch(1)
            shuffle(1)
            put(i0 + 1, 1)
            fetch(i0 + 3, 1)

        wait_put(0)
        wait_fetch(0)
        shuffle(0)
        put(A_PER_W - 2, 0)
        wait_put(1)
        wait_fetch(1)
        shuffle(1)
        put(A_PER_W - 1, 1)
        wait_put(0)
        wait_put(1)

        @pl.when(wid == 0)
        def _tail():
            pltpu.sync_copy(tail_hbm, tail_v)
            pltpu.sync_copy(tail_v, wpad_hbm.at[pl.ds(TAIL0 * D, D * D)])

    return detile


@functools.lru_cache(maxsize=None)
def _build_gather():
    mesh = plsc.VectorSubcoreMesh(core_axis_name="c", subcore_axis_name="s")
    CHUNK = TILE_T * TLEN
    TSLAB = 8 * SBLK * 1024

    @functools.partial(
        pl.kernel,
        out_type=jax.ShapeDtypeStruct((TLEN, 8, SBLK, 8 * TILE_T),
                                      jnp.float32),
        mesh=mesh,
        compiler_params=pltpu.CompilerParams(
            use_tc_tiling_on_sc=False, needs_layout_passes=False,
            disable_bounds_checks=True),
        scratch_types=[
            pltpu.VMEM((CHUNK,), jnp.int32),
            pltpu.VMEM((2, TILE_T), jnp.int32),
            pltpu.VMEM((2, TILE_T, D), jnp.float32),
            pltpu.VMEM((2, 8, 1, 8 * TILE_T), jnp.float32),
            pltpu.SemaphoreType.DMA,
            pltpu.SemaphoreType.DMA,
            pltpu.SemaphoreType.DMA,
            pltpu.SemaphoreType.DMA,
        ],
    )
    def gather(tok_hbm, wpad_hbm, out_hbm, tok_v, idx_v, rows_v, dst_v,
               gsem0, gsem1, osem0, osem1):
        gsems = (gsem0, gsem1)
        osems = (osem0, osem1)
        sblk = lax.axis_index("s") * NUM_CORES + lax.axis_index("c")
        C0, C1 = _idx_consts()
        lane = lax.iota(jnp.int32, 16)

        pltpu.sync_copy(tok_hbm.at[pl.ds(sblk * CHUNK, CHUNK)], tok_v)

        def prep(t, b):
            for m8 in range(8):
                v = plsc.load_gather(
                    tok_v, [_splat(t) + (lane + 16 * m8) * TLEN])
                idx_v[b, pl.ds(16 * m8, 16)] = v
            pltpu.async_copy(wpad_hbm.at[idx_v.at[b]], rows_v.at[b],
                             gsems[b])

        def wait_gather(b):
            pltpu.make_async_copy(
                wpad_hbm.at[pl.ds(0, TILE_T)], rows_v.at[b],
                gsems[b]).wait()

        diag = [(lane + k) & 15 for k in range(16)]
        dhi = [d >> 3 for d in diag]
        dlo = [(d & 7) * TILE_T + lane for d in diag]

        def shuffle(b):
            @plsc.parallel_loop(0, TILE_T // 16)
            def _sblk(m):
                sr0 = m * 16
                ss = _splat(sr0) + lane
                for f0 in range(0, D, 16):
                    for k in range(16):
                        v = plsc.load_gather(
                            rows_v.at[b], [ss, _splat(f0) + diag[k]])
                        plsc.store_scatter(
                            dst_v.at[b],
                            [_splat(f0 >> 3) + dhi[k], _splat(0),
                             _splat(sr0) + dlo[k]], v)

        def put(t, b):
            pltpu.async_copy(
                dst_v.at[b],
                out_hbm.at[t, :, pl.ds(sblk, 1)],
                osems[b])

        def wait_put(b):
            pltpu.make_async_copy(
                dst_v.at[b], out_hbm.at[0, :, pl.ds(0, 1)],
                osems[b]).wait()

        prep(0, 0)
        prep(1, 1)
        wait_gather(0)
        shuffle(0)
        put(0, 0)
        prep(2, 0)
        wait_gather(1)
        shuffle(1)
        put(1, 1)
        prep(3, 1)

        @pl.loop(1, TLEN // 2 - 1)
        def _pair(k):
            t0 = 2 * k
            wait_put(0)
            wait_gather(0)
            shuffle(0)
            put(t0, 0)
            prep(t0 + 2, 0)
            wait_put(1)
            wait_gather(1)
            shuffle(1)
            put(t0 + 1, 1)
            prep(t0 + 3, 1)

        wait_put(0)
        wait_gather(0)
        shuffle(0)
        put(TLEN - 2, 0)
        wait_put(1)
        wait_gather(1)
        shuffle(1)
        put(TLEN - 1, 1)
        wait_put(0)
        wait_put(1)

    return gather


def kernel(tokens, W_E):
    tok_flat = tokens.astype(jnp.int32).reshape(-1)
    w_t = W_E.T
    tail = W_E[TAIL0:, :].reshape(-1)
    w_pad = _build_detile()(w_t, tail)
    w_pad2d = w_pad.reshape(V_PAD, D)
    out1 = _build_gather()(tok_flat, w_pad2d)
    out5 = out1.reshape(TLEN, 8, SBLK, 8, TILE_T)
    return out5.transpose(2, 4, 0, 1, 3).reshape(SEQ, TLEN, D)


# speedup vs baseline: 1.1058x; 1.1058x over previous
import functools

import jax
import jax.numpy as jnp
from jax import lax
from jax.experimental import pallas as pl
from jax.experimental.pallas import tpu as pltpu
from jax.experimental.pallas import tpu_sc as plsc

NUM_CORES = 2
NUM_SUBCORES = 16
NUM_WORKERS = NUM_CORES * NUM_SUBCORES

V = 1000000
D = 64
TILE_T = 128
NTB = (V + TILE_T - 1) // TILE_T
V_PAD = NTB * TILE_T
FULL_BLOCKS = V // TILE_T
TAIL0 = FULL_BLOCKS * TILE_T

A_PER_W = 246
A_START = 245

SEQ = 4096
TLEN = 200
B_TOTAL = SEQ * TLEN
SBLK = SEQ // TILE_T


def _idx_consts():
    i = lax.iota(jnp.int32, 16)
    return i >> 3, i & 7


def _splat(x):
    return jnp.full((16,), x, jnp.int32)


@functools.lru_cache(maxsize=None)
def _build_detile():
    mesh = plsc.VectorSubcoreMesh(core_axis_name="c", subcore_axis_name="s")

    @functools.partial(
        pl.kernel,
        out_type=jax.ShapeDtypeStruct((V_PAD * D,), jnp.float32),
        mesh=mesh,
        compiler_params=pltpu.CompilerParams(
            use_tc_tiling_on_sc=True, needs_layout_passes=False,
            disable_bounds_checks=True),
        scratch_types=[
            pltpu.VMEM((2, D, TILE_T), jnp.float32),
            pltpu.VMEM((TILE_T * D,), jnp.float32),
            pltpu.VMEM((TILE_T * D,), jnp.float32),
            pltpu.VMEM((D * D,), jnp.float32),
            pltpu.SemaphoreType.DMA,
            pltpu.SemaphoreType.DMA,
            pltpu.SemaphoreType.DMA,
            pltpu.SemaphoreType.DMA,
        ],
    )
    def detile(wt_hbm, tail_hbm, wpad_hbm, src_v, dst_v0, dst_v1, tail_v,
               gsem0, gsem1, osem0, osem1):
        gsems = (gsem0, gsem1)
        osems = (osem0, osem1)
        dsts = (dst_v0, dst_v1)
        wid = lax.axis_index("s") * NUM_CORES + lax.axis_index("c")
        start = wid * A_START
        C0, C1 = _idx_consts()

        def blk(i):
            return jnp.minimum(start + i, FULL_BLOCKS - 1)

        def fetch(i, b):
            pltpu.async_copy(
                wt_hbm.at[:, pl.ds(blk(i) * TILE_T, TILE_T)],
                src_v.at[b], gsems[b])

        def wait_fetch(b):
            pltpu.make_async_copy(
                wt_hbm.at[:, pl.ds(0, TILE_T)], src_v.at[b],
                gsems[b]).wait()

        lane = lax.iota(jnp.int32, 16)
        diag = [(lane + k) & 15 for k in range(16)]
        dstc = [d * D + lane for d in diag]

        def shuffle(b):
            @plsc.parallel_loop(0, TILE_T // 16)
            def _tblk(m):
                t0 = m * 16
                for f0 in range(0, D, 16):
                    ff = _splat(f0) + lane
                    for k in range(16):
                        v = plsc.load_gather(
                            src_v.at[b], [ff, _splat(t0) + diag[k]])
                        plsc.store_scatter(
                            dsts[b], [_splat(t0 * D + f0) + dstc[k]], v)

        def put(i, b):
            pltpu.async_copy(
                dsts[b],
                wpad_hbm.at[pl.ds(blk(i) * (TILE_T * D), TILE_T * D)],
                osems[b])

        def wait_put(b):
            pltpu.make_async_copy(
                dsts[b], wpad_hbm.at[pl.ds(0, TILE_T * D)],
                osems[b]).wait()

        fetch(0, 0)
        fetch(1, 1)
        wait_fetch(0)
        shuffle(0)
        put(0, 0)
        fetch(2, 0)
        wait_fetch(1)
        shuffle(1)
        put(1, 1)
        fetch(3, 1)

        @pl.loop(1, A_PER_W // 2 - 1)
        def _pair(k):
            i0 = 2 * k
            wait_put(0)
            wait_fetch(0)
            shuffle(0)
            put(i0, 0)
            fetch(i0 + 2, 0)
            wait_put(1)
            wait_fetch(1)
            shuffle(1)
            put(i0 + 1, 1)
            fetch(i0 + 3, 1)

        wait_put(0)
        wait_fetch(0)
        shuffle(0)
        put(A_PER_W - 2, 0)
        wait_put(1)
        wait_fetch(1)
        shuffle(1)
        put(A_PER_W - 1, 1)
        wait_put(0)
        wait_put(1)

        @pl.when(wid == 0)
        def _tail():
            pltpu.sync_copy(tail_hbm, tail_v)
            pltpu.sync_copy(tail_v, wpad_hbm.at[pl.ds(TAIL0 * D, D * D)])

    return detile


@functools.lru_cache(maxsize=None)
def _build_gather():
    mesh = plsc.VectorSubcoreMesh(core_axis_name="c", subcore_axis_name="s")
    CHUNK = TILE_T * TLEN
    TSLAB = 8 * SBLK * 1024

    @functools.partial(
        pl.kernel,
        out_type=jax.ShapeDtypeStruct((TLEN * D * SEQ,), jnp.float32),
        mesh=mesh,
        compiler_params=pltpu.CompilerParams(
            use_tc_tiling_on_sc=False, needs_layout_passes=False,
            disable_bounds_checks=True),
        scratch_types=[
            pltpu.VMEM((CHUNK,), jnp.int32),
            pltpu.VMEM((2, TILE_T), jnp.int32),
            pltpu.VMEM((2, TILE_T, D), jnp.float32),
            pltpu.VMEM((2, D * TILE_T), jnp.float32),
            pltpu.SemaphoreType.DMA,
            pltpu.SemaphoreType.DMA,
            pltpu.SemaphoreType.DMA,
            pltpu.SemaphoreType.DMA,
        ],
    )
    def gather(tok_hbm, wpad_hbm, out_hbm, tok_v, idx_v, rows_v, dst_v,
               gsem0, gsem1, osem0, osem1):
        gsems = (gsem0, gsem1)
        osems = (osem0, osem1)
        sblk = lax.axis_index("s") * NUM_CORES + lax.axis_index("c")
        C0, C1 = _idx_consts()
        lane = lax.iota(jnp.int32, 16)

        pltpu.sync_copy(tok_hbm.at[pl.ds(sblk * CHUNK, CHUNK)], tok_v)

        def prep(t, b):
            for m8 in range(8):
                v = plsc.load_gather(
                    tok_v, [_splat(t) + (lane + 16 * m8) * TLEN])
                idx_v[b, pl.ds(16 * m8, 16)] = v
            pltpu.async_copy(wpad_hbm.at[idx_v.at[b]], rows_v.at[b],
                             gsems[b])

        def wait_gather(b):
            pltpu.make_async_copy(
                wpad_hbm.at[pl.ds(0, TILE_T)], rows_v.at[b],
                gsems[b]).wait()

        diag = [(lane + k) & 15 for k in range(16)]
        dstc = [(d >> 3) * 1024 + (d & 7) * TILE_T + lane for d in diag]

        def shuffle(b):
            @plsc.parallel_loop(0, TILE_T // 16)
            def _sblk(m):
                sr0 = m * 16
                ss = _splat(sr0) + lane
                for f0 in range(0, D, 16):
                    for k in range(16):
                        v = plsc.load_gather(
                            rows_v.at[b], [ss, _splat(f0) + diag[k]])
                        plsc.store_scatter(
                            dst_v.at[b],
                            [_splat((f0 >> 3) * 1024 + sr0) + dstc[k]], v)

        def put(t, b):
            base = t * TSLAB + sblk * 1024
            for dgrp in range(8):
                pltpu.async_copy(
                    dst_v.at[b, pl.ds(dgrp * 1024, 1024)],
                    out_hbm.at[pl.ds(base + dgrp * (SBLK * 1024), 1024)],
                    osems[b])

        def wait_put(b):
            pltpu.make_async_copy(
                dst_v.at[b], out_hbm.at[pl.ds(0, D * TILE_T)],
                osems[b]).wait()

        prep(0, 0)
        prep(1, 1)
        wait_gather(0)
        shuffle(0)
        put(0, 0)
        prep(2, 0)
        wait_gather(1)
        shuffle(1)
        put(1, 1)
        prep(3, 1)

        @pl.loop(1, TLEN // 2 - 1)
        def _pair(k):
            t0 = 2 * k
            wait_put(0)
            wait_gather(0)
            shuffle(0)
            put(t0, 0)
            prep(t0 + 2, 0)
            wait_put(1)
            wait_gather(1)
            shuffle(1)
            put(t0 + 1, 1)
            prep(t0 + 3, 1)

        wait_put(0)
        wait_gather(0)
        shuffle(0)
        put(TLEN - 2, 0)
        wait_put(1)
        wait_gather(1)
        shuffle(1)
        put(TLEN - 1, 1)
        wait_put(0)
        wait_put(1)

    return gather


def kernel(tokens, W_E):
    tok_flat = tokens.astype(jnp.int32).reshape(-1)
    w_t = W_E.T
    tail = W_E[TAIL0:, :].reshape(-1)
    w_pad = _build_detile()(w_t, tail)
    w_pad2d = w_pad.reshape(V_PAD, D)
    out1 = _build_gather()(tok_flat, w_pad2d)
    out5 = out1.reshape(TLEN, 8, SBLK, 8, TILE_T)
    return out5.transpose(2, 4, 0, 1, 3).reshape(SEQ, TLEN, D)
